# baseline (speedup 1.0000x reference)
import jax
import jax.numpy as jnp
from jax.experimental import pallas as pl
from jax.experimental.pallas import tpu as pltpu
from jax.experimental.pallas import tpu_sc as plsc

D_MODEL = 768
MAX_REL = 32
SEQ = 512
G_ROWS = 1024
N_SHIFT = 8
BI = 8
N_WORKERS = 32
ROWS_PER_W = (N_SHIFT * G_ROWS) // N_WORKERS
CHUNK = 128


def _sc_gather_body(tab_hbm, idx_hbm, g2_hbm, idx_v, rows_v, sem):
    wid = jax.lax.axis_index("s") * 2 + jax.lax.axis_index("c")
    for c in range(ROWS_PER_W // CHUNK):
        base = wid * ROWS_PER_W + c * CHUNK
        pltpu.sync_copy(idx_hbm.at[pl.ds(base, CHUNK)], idx_v)
        pltpu.async_copy(tab_hbm.at[idx_v], rows_v, sem).wait()
        pltpu.sync_copy(rows_v, g2_hbm.at[pl.ds(base, CHUNK)])


def _add_body(x_ref, g_ref, o_ref):
    i0 = pl.program_id(0) * BI
    base = pl.multiple_of(504 - i0, 8)
    xv = x_ref[:]
    for r in range(BI):
        o_ref[r] = xv + g_ref[7 - r, pl.ds(base, SEQ), :]


def kernel(x, table):
    x2d = x.reshape(SEQ, D_MODEL)

    r = jnp.arange(N_SHIFT * G_ROWS, dtype=jnp.int32)
    s, k = r // G_ROWS, r % G_ROWS
    idx = (jnp.clip(511 - k - s, -MAX_REL, MAX_REL) + MAX_REL).astype(jnp.int32)

    g2_flat = pl.kernel(
        _sc_gather_body,
        out_type=jax.ShapeDtypeStruct((N_SHIFT * G_ROWS, D_MODEL), jnp.float32),
        mesh=plsc.VectorSubcoreMesh(core_axis_name="c", subcore_axis_name="s"),
        scratch_types=[
            pltpu.VMEM((CHUNK,), jnp.int32),
            pltpu.VMEM((CHUNK, D_MODEL), jnp.float32),
            pltpu.SemaphoreType.DMA,
        ],
    )(table, idx)
    g2 = g2_flat.reshape(N_SHIFT, G_ROWS, D_MODEL)

    out = pl.pallas_call(
        _add_body,
        grid=(SEQ // BI,),
        in_specs=[
            pl.BlockSpec((SEQ, D_MODEL), lambda i: (0, 0)),
            pl.BlockSpec((N_SHIFT, G_ROWS, D_MODEL), lambda i: (0, 0, 0)),
        ],
        out_specs=pl.BlockSpec((BI, SEQ, D_MODEL), lambda i: (i, 0, 0)),
        out_shape=jax.ShapeDtypeStruct((SEQ, SEQ, D_MODEL), jnp.float32),
        compiler_params=pltpu.CompilerParams(
            dimension_semantics=("arbitrary",),
        ),
    )(x2d, g2)
    return out
